# baseline (speedup 1.0000x reference)
import jax
import jax.numpy as jnp
from jax import lax
from jax.experimental import pallas as pl
from jax.experimental.pallas import tpu as pltpu
from jax.experimental.pallas import tpu_sc as plsc

_N = 10000
_E = 320000
_D = 128
_L = 16
_NS = 16
_NB = _E // 128
_NBW = _NB // _NS + (1 if _NB % _NS else 0)


def _prep_body(h_ref, w_ref, ij_ref, b_ref, t_ref, sib_ref, enum_ref, pi_ref):
    w2 = w_ref[...].reshape(2, _D)
    dn = (((1,), (1,)), ((), ()))
    d = lax.dot_general(h_ref[...], w2, dn,
                        preferred_element_type=jnp.float32,
                        precision=jax.lax.Precision.HIGHEST)
    t_ref[...] = d
    ii = ij_ref[0]
    jj = ij_ref[1]
    di = lax.dot_general(h_ref[pl.ds(ii, 1), :], w2, dn,
                         preferred_element_type=jnp.float32,
                         precision=jax.lax.Precision.HIGHEST)
    dj = lax.dot_general(h_ref[pl.ds(jj, 1), :], w2, dn,
                         preferred_element_type=jnp.float32,
                         precision=jax.lax.Precision.HIGHEST)
    sib = di[:, 0] + b_ref[0]
    x0 = sib + dj[:, 1]
    enum = jnp.where(x0 >= 0, x0, 0.2 * x0)
    sib_ref[...] = jnp.broadcast_to(sib, (_L,))
    enum_ref[...] = jnp.broadcast_to(enum, (_L,))
    pi_ref[...] = jnp.full((_L,), ii, jnp.int32)


def _edge_scan_body(t_hbm, g_hbm, pi_hbm, sib_hbm, enum_hbm, out_hbm,
                    t_v, g_v, pi_v, sib_v, enum_v, stage_v, parts_sh, parts_v):
    s = lax.axis_index("s")
    b0 = (_NB * s) // _NS
    b1 = (_NB * (s + 1)) // _NS
    pltpu.sync_copy(pi_hbm, pi_v)
    pltpu.sync_copy(sib_hbm, sib_v)
    pltpu.sync_copy(enum_hbm, enum_v)
    pltpu.sync_copy(t_hbm, t_v)
    pltpu.sync_copy(g_hbm.at[pl.ds(b0, _NBW)], g_v)
    pi = pi_v[...]
    sib = sib_v[...]

    def body(q, acc):
        for m in range(8):
            src = g_v[q, 0, pl.ds(m * _L, _L)]
            dst = g_v[q, 1, pl.ds(m * _L, _L)]
            tv = plsc.load_gather(t_v, [2 * dst + 1])
            x = sib + tv
            lr = jnp.where(x >= 0, x, 0.2 * x)
            acc = acc + jnp.where(src == pi, lr, 0.0)
        return acc

    acc = lax.fori_loop(0, b1 - b0, body, jnp.zeros((_L,), jnp.float32))
    stage_v[...] = acc
    pltpu.sync_copy(stage_v, parts_sh.at[s])
    plsc.subcore_barrier()

    @pl.when(s == 0)
    def _():
        pltpu.sync_copy(parts_sh, parts_v)
        tot16 = jnp.zeros((_L,), jnp.float32)
        for r in range(_NS):
            tot16 = tot16 + parts_v[r]
        total = jnp.sum(tot16)
        stage_v[...] = enum_v[...] / total
        pltpu.sync_copy(stage_v, out_hbm)


def kernel(g, h, i, j, W, b):
    ij = jnp.stack([jnp.asarray(i, jnp.int32), jnp.asarray(j, jnp.int32)])
    t, sib, enum, pi = pl.pallas_call(
        _prep_body,
        in_specs=[
            pl.BlockSpec(memory_space=pltpu.VMEM),
            pl.BlockSpec(memory_space=pltpu.VMEM),
            pl.BlockSpec(memory_space=pltpu.SMEM),
            pl.BlockSpec(memory_space=pltpu.SMEM),
        ],
        out_shape=[
            jax.ShapeDtypeStruct((_N, 2), jnp.float32),
            jax.ShapeDtypeStruct((_L,), jnp.float32),
            jax.ShapeDtypeStruct((_L,), jnp.float32),
            jax.ShapeDtypeStruct((_L,), jnp.int32),
        ],
    )(h, W, ij, b)
    mesh = plsc.VectorSubcoreMesh(core_axis_name="c", subcore_axis_name="s",
                                  num_cores=2, num_subcores=_NS)
    scan_call = pl.kernel(
        _edge_scan_body,
        out_type=jax.ShapeDtypeStruct((_L,), jnp.float32),
        mesh=mesh,
        compiler_params=pltpu.CompilerParams(
            needs_layout_passes=False, use_tc_tiling_on_sc=False),
        scratch_types=[
            pltpu.VMEM((_N * 2,), jnp.float32),
            pltpu.VMEM((_NBW, 2, 128), jnp.int32),
            pltpu.VMEM((_L,), jnp.int32),
            pltpu.VMEM((_L,), jnp.float32),
            pltpu.VMEM((_L,), jnp.float32),
            pltpu.VMEM((_L,), jnp.float32),
            pltpu.VMEM_SHARED((_NS, _L), jnp.float32),
            pltpu.VMEM((_NS, _L), jnp.float32),
        ],
    )
    gview = g.reshape(_NB, 128, 2).transpose(0, 2, 1)
    scan = scan_call(t.reshape(_N * 2), gview, pi, sib, enum)
    return scan[0:1]


# speedup vs baseline: 1.2443x; 1.2443x over previous
import jax
import jax.numpy as jnp
from jax import lax
from jax.experimental import pallas as pl
from jax.experimental.pallas import tpu as pltpu
from jax.experimental.pallas import tpu_sc as plsc

_N = 10000
_E = 320000
_D = 128
_L = 16
_NS = 16
_NB = _E // 128
_NBW = _NB // _NS + (1 if _NB % _NS else 0)


def _prep_body(h_ref, w_ref, ij_ref, b_ref, t_ref, sib_ref, enum_ref, pi_ref):
    w2 = w_ref[...].reshape(2, _D)
    dn = (((1,), (1,)), ((), ()))
    d = lax.dot_general(h_ref[...], w2, dn,
                        preferred_element_type=jnp.float32,
                        precision=jax.lax.Precision.HIGHEST)
    t_ref[...] = jnp.transpose(d)[1, :]
    ii = ij_ref[0]
    jj = ij_ref[1]
    di = lax.dot_general(h_ref[pl.ds(ii, 1), :], w2, dn,
                         preferred_element_type=jnp.float32,
                         precision=jax.lax.Precision.HIGHEST)
    dj = lax.dot_general(h_ref[pl.ds(jj, 1), :], w2, dn,
                         preferred_element_type=jnp.float32,
                         precision=jax.lax.Precision.HIGHEST)
    sib = di[:, 0] + b_ref[0]
    x0 = sib + dj[:, 1]
    enum = jnp.where(x0 >= 0, x0, 0.2 * x0)
    sib_ref[...] = jnp.broadcast_to(sib, (_L,))
    enum_ref[...] = jnp.broadcast_to(enum, (_L,))
    pi_ref[...] = jnp.full((_L,), ii, jnp.int32)


def _edge_scan_body(t_hbm, g_hbm, pi_hbm, sib_hbm, enum_hbm, out_hbm,
                    t_v, g_v, pi_v, sib_v, enum_v, stage_v, parts_sh, parts_v):
    s = lax.axis_index("s")
    b0 = (_NB * s) // _NS
    b1 = (_NB * (s + 1)) // _NS
    pltpu.sync_copy(pi_hbm, pi_v)
    pltpu.sync_copy(sib_hbm, sib_v)
    pltpu.sync_copy(enum_hbm, enum_v)
    pltpu.sync_copy(t_hbm, t_v)
    pltpu.sync_copy(g_hbm.at[pl.ds(b0, _NBW)], g_v)
    pi = pi_v[...]
    sib = sib_v[...]

    def body(q, acc):
        for m in range(8):
            src = g_v[q, 0, pl.ds(m * _L, _L)]
            dst = g_v[q, 1, pl.ds(m * _L, _L)]
            tv = plsc.load_gather(t_v, [dst])
            x = sib + tv
            lr = jnp.where(x >= 0, x, 0.2 * x)
            acc = acc + jnp.where(src == pi, lr, 0.0)
        return acc

    acc = lax.fori_loop(0, b1 - b0, body, jnp.zeros((_L,), jnp.float32))
    stage_v[...] = acc
    pltpu.sync_copy(stage_v, parts_sh.at[s])
    plsc.subcore_barrier()

    @pl.when(s == 0)
    def _():
        pltpu.sync_copy(parts_sh, parts_v)
        tot16 = jnp.zeros((_L,), jnp.float32)
        for r in range(_NS):
            tot16 = tot16 + parts_v[r]
        total = jnp.sum(tot16)
        stage_v[...] = enum_v[...] / total
        pltpu.sync_copy(stage_v, out_hbm)


def kernel(g, h, i, j, W, b):
    ij = jnp.stack([jnp.asarray(i, jnp.int32), jnp.asarray(j, jnp.int32)])
    t, sib, enum, pi = pl.pallas_call(
        _prep_body,
        in_specs=[
            pl.BlockSpec(memory_space=pltpu.VMEM),
            pl.BlockSpec(memory_space=pltpu.VMEM),
            pl.BlockSpec(memory_space=pltpu.SMEM),
            pl.BlockSpec(memory_space=pltpu.SMEM),
        ],
        out_shape=[
            jax.ShapeDtypeStruct((_N,), jnp.float32),
            jax.ShapeDtypeStruct((_L,), jnp.float32),
            jax.ShapeDtypeStruct((_L,), jnp.float32),
            jax.ShapeDtypeStruct((_L,), jnp.int32),
        ],
    )(h, W, ij, b)
    mesh = plsc.VectorSubcoreMesh(core_axis_name="c", subcore_axis_name="s",
                                  num_cores=2, num_subcores=_NS)
    scan_call = pl.kernel(
        _edge_scan_body,
        out_type=jax.ShapeDtypeStruct((_L,), jnp.float32),
        mesh=mesh,
        compiler_params=pltpu.CompilerParams(
            needs_layout_passes=False, use_tc_tiling_on_sc=False),
        scratch_types=[
            pltpu.VMEM((_N,), jnp.float32),
            pltpu.VMEM((_NBW, 2, 128), jnp.int32),
            pltpu.VMEM((_L,), jnp.int32),
            pltpu.VMEM((_L,), jnp.float32),
            pltpu.VMEM((_L,), jnp.float32),
            pltpu.VMEM((_L,), jnp.float32),
            pltpu.VMEM_SHARED((_NS, _L), jnp.float32),
            pltpu.VMEM((_NS, _L), jnp.float32),
        ],
    )
    gview = g.reshape(_NB, 128, 2).transpose(0, 2, 1)
    scan = scan_call(t, gview, pi, sib, enum)
    return scan[0:1]


# speedup vs baseline: 1.2516x; 1.0059x over previous
import jax
import jax.numpy as jnp
from jax import lax
from jax.experimental import pallas as pl
from jax.experimental.pallas import tpu as pltpu
from jax.experimental.pallas import tpu_sc as plsc

_N = 10000
_E = 320000
_D = 128
_L = 16
_NS = 16
_NB = _E // 128
_NBW = _NB // _NS + (1 if _NB % _NS else 0)


def _prep_body(h_ref, w_ref, ij_ref, b_ref, t_ref, sib_ref, enum_ref, pi_ref):
    w2 = w_ref[...].reshape(2, _D)
    dn = (((1,), (1,)), ((), ()))
    d = lax.dot_general(h_ref[...], w2, dn,
                        preferred_element_type=jnp.float32,
                        precision=jax.lax.Precision.HIGHEST)
    t_ref[...] = jnp.transpose(d)[1, :]
    ii = ij_ref[0]
    jj = ij_ref[1]
    di = lax.dot_general(h_ref[pl.ds(ii, 1), :], w2, dn,
                         preferred_element_type=jnp.float32,
                         precision=jax.lax.Precision.HIGHEST)
    dj = lax.dot_general(h_ref[pl.ds(jj, 1), :], w2, dn,
                         preferred_element_type=jnp.float32,
                         precision=jax.lax.Precision.HIGHEST)
    sib = di[:, 0] + b_ref[0]
    x0 = sib + dj[:, 1]
    enum = jnp.where(x0 >= 0, x0, 0.2 * x0)
    sib_ref[...] = jnp.broadcast_to(sib, (_L,))
    enum_ref[...] = jnp.broadcast_to(enum, (_L,))
    pi_ref[...] = jnp.full((_L,), ii, jnp.int32)


def _edge_scan_body(t_hbm, g_hbm, pi_hbm, sib_hbm, enum_hbm, out_hbm,
                    t_v, g_v, pi_v, sib_v, enum_v, stage_v, parts_sh, parts_v):
    s = lax.axis_index("s")
    b0 = (_NB * s) // _NS
    b1 = (_NB * (s + 1)) // _NS
    pltpu.sync_copy(pi_hbm, pi_v)
    pltpu.sync_copy(sib_hbm, sib_v)
    pltpu.sync_copy(enum_hbm, enum_v)
    pltpu.sync_copy(t_hbm, t_v)
    pltpu.sync_copy(g_hbm.at[pl.ds(b0, _NBW)], g_v)
    pi = pi_v[...]
    sib = sib_v[...]

    def body(q, accs):
        out = []
        for m in range(8):
            src = g_v[q, 0, pl.ds(m * _L, _L)]
            dst = g_v[q, 1, pl.ds(m * _L, _L)]
            tv = plsc.load_gather(t_v, [dst])
            x = sib + tv
            lr = jnp.maximum(x, 0.2 * x)
            out.append(accs[m] + jnp.where(src == pi, lr, 0.0))
        return tuple(out)

    z16 = jnp.zeros((_L,), jnp.float32)
    accs = lax.fori_loop(0, b1 - b0, body, (z16,) * 8)
    acc = ((accs[0] + accs[1]) + (accs[2] + accs[3])) + \
          ((accs[4] + accs[5]) + (accs[6] + accs[7]))
    stage_v[...] = acc
    pltpu.sync_copy(stage_v, parts_sh.at[s])
    plsc.subcore_barrier()

    @pl.when(s == 0)
    def _():
        pltpu.sync_copy(parts_sh, parts_v)
        tot16 = jnp.zeros((_L,), jnp.float32)
        for r in range(_NS):
            tot16 = tot16 + parts_v[r]
        total = jnp.sum(tot16)
        stage_v[...] = enum_v[...] / total
        pltpu.sync_copy(stage_v, out_hbm)


def kernel(g, h, i, j, W, b):
    ij = jnp.stack([jnp.asarray(i, jnp.int32), jnp.asarray(j, jnp.int32)])
    t, sib, enum, pi = pl.pallas_call(
        _prep_body,
        in_specs=[
            pl.BlockSpec(memory_space=pltpu.VMEM),
            pl.BlockSpec(memory_space=pltpu.VMEM),
            pl.BlockSpec(memory_space=pltpu.SMEM),
            pl.BlockSpec(memory_space=pltpu.SMEM),
        ],
        out_shape=[
            jax.ShapeDtypeStruct((_N,), jnp.float32),
            jax.ShapeDtypeStruct((_L,), jnp.float32),
            jax.ShapeDtypeStruct((_L,), jnp.float32),
            jax.ShapeDtypeStruct((_L,), jnp.int32),
        ],
    )(h, W, ij, b)
    mesh = plsc.VectorSubcoreMesh(core_axis_name="c", subcore_axis_name="s",
                                  num_cores=2, num_subcores=_NS)
    scan_call = pl.kernel(
        _edge_scan_body,
        out_type=jax.ShapeDtypeStruct((_L,), jnp.float32),
        mesh=mesh,
        compiler_params=pltpu.CompilerParams(
            needs_layout_passes=False, use_tc_tiling_on_sc=False),
        scratch_types=[
            pltpu.VMEM((_N,), jnp.float32),
            pltpu.VMEM((_NBW, 2, 128), jnp.int32),
            pltpu.VMEM((_L,), jnp.int32),
            pltpu.VMEM((_L,), jnp.float32),
            pltpu.VMEM((_L,), jnp.float32),
            pltpu.VMEM((_L,), jnp.float32),
            pltpu.VMEM_SHARED((_NS, _L), jnp.float32),
            pltpu.VMEM((_NS, _L), jnp.float32),
        ],
    )
    gview = g.reshape(_NB, 128, 2).transpose(0, 2, 1)
    scan = scan_call(t, gview, pi, sib, enum)
    return scan[0:1]
